# Initial kernel scaffold; baseline (speedup 1.0000x reference)
import jax
import jax.numpy as jnp
from jax.experimental import pallas as pl


def kernel(sequence, segment_label, token_table, segment_table):
    raise NotImplementedError("write your pallas kernel here")


# speedup vs baseline: 4.4944x; 4.4944x over previous
import functools

import numpy as np
import jax
import jax.numpy as jnp
from jax import lax
from jax.experimental import pallas as pl
from jax.experimental.pallas import tpu as pltpu
from jax.experimental.pallas import tpu_sc as plsc

VOCAB = 100000
EMBED = 64
BATCH = 1024
SEQLEN = 200
MAX_LEN = 512

NC = 2
NS = 16
NW = NC * NS
TOT = BATCH * SEQLEN
PER_W = TOT // NW
CHUNK = 256
NSUB = CHUNK // 128
NCH = PER_W // CHUNK


def _positional_encoding(max_len, d_model):
    position = np.arange(max_len, dtype=np.float32)[:, None]
    div_term = np.exp(
        np.arange(0, d_model, 2, dtype=np.float32) * -(np.log(10000.0) / d_model)
    )
    pe = np.zeros((max_len, d_model), dtype=np.float32)
    pe[:, 0::2] = np.sin(position * div_term)
    pe[:, 1::2] = np.cos(position * div_term)
    return pe


_PE = jnp.asarray(_positional_encoding(MAX_LEN, EMBED)[:SEQLEN])

_mesh = plsc.VectorSubcoreMesh(
    core_axis_name="c", subcore_axis_name="s", num_cores=NC, num_subcores=NS
)


@functools.partial(
    pl.kernel,
    out_type=jax.ShapeDtypeStruct((TOT, EMBED), jnp.float32),
    mesh=_mesh,
    scratch_types=[
        pltpu.VMEM((NSUB, 128), jnp.int32),
        pltpu.VMEM((NSUB, 128), jnp.int32),
        pltpu.VMEM((NSUB, 128), jnp.int32),
        pltpu.VMEM((CHUNK, EMBED), jnp.float32),
        pltpu.SemaphoreType.DMA,
    ],
    compiler_params=pltpu.CompilerParams(use_tc_tiling_on_sc=False),
)
def _embed_kernel(seq_hbm, lbl_hbm, tok_hbm, c_hbm, out_hbm,
                  sidx, lidx, cidx, dbuf, sem):
    wid = lax.axis_index("s") * NC + lax.axis_index("c")
    wbase = wid * PER_W

    def chunk_body(ci, carry):
        base = wbase + ci * CHUNK
        brow = wid * (PER_W // 128) + ci * NSUB
        pltpu.sync_copy(seq_hbm.at[pl.ds(brow, NSUB)], sidx)
        pltpu.sync_copy(lbl_hbm.at[pl.ds(brow, NSUB)], lidx)
        for j in range(CHUNK // 16):
            s, c = divmod(j, 8)
            lane = lax.iota(jnp.int32, 16) + (base + j * 16)
            pos = lax.rem(lane, SEQLEN)
            cidx[s, pl.ds(c * 16, 16)] = pos * 3 + lidx[s, pl.ds(c * 16, 16)]
        cps = [
            pltpu.async_copy(
                tok_hbm.at[sidx.at[s]], dbuf.at[pl.ds(s * 128, 128)], sem
            )
            for s in range(NSUB)
        ]
        for cp in cps:
            cp.wait()
        cps = [
            pltpu.async_copy(
                c_hbm.at[cidx.at[s]], dbuf.at[pl.ds(s * 128, 128)], sem, add=True
            )
            for s in range(NSUB)
        ]
        for cp in cps:
            cp.wait()
        pltpu.sync_copy(dbuf, out_hbm.at[pl.ds(base, CHUNK)])
        return carry

    lax.fori_loop(0, NCH, chunk_body, 0)


def kernel(sequence, segment_label, token_table, segment_table):
    seq2d = sequence.reshape(TOT // 128, 128).astype(jnp.int32)
    lbl2d = segment_label.reshape(TOT // 128, 128).astype(jnp.int32)
    c_tab = (_PE[:, None, :] + segment_table[None, :, :]).reshape(3 * SEQLEN, EMBED)
    out = _embed_kernel(seq2d, lbl2d, token_table, c_tab)
    return out.reshape(BATCH, SEQLEN, EMBED)


# speedup vs baseline: 4.9135x; 1.0933x over previous
import functools

import numpy as np
import jax
import jax.numpy as jnp
from jax import lax
from jax.experimental import pallas as pl
from jax.experimental.pallas import tpu as pltpu
from jax.experimental.pallas import tpu_sc as plsc

VOCAB = 100000
EMBED = 64
BATCH = 1024
SEQLEN = 200
MAX_LEN = 512

NC = 2
NS = 16
NW = NC * NS
TOT = BATCH * SEQLEN
PER_W = TOT // NW
CHUNK = 640
NSUB = CHUNK // 128
NCH = PER_W // CHUNK


def _positional_encoding(max_len, d_model):
    position = np.arange(max_len, dtype=np.float32)[:, None]
    div_term = np.exp(
        np.arange(0, d_model, 2, dtype=np.float32) * -(np.log(10000.0) / d_model)
    )
    pe = np.zeros((max_len, d_model), dtype=np.float32)
    pe[:, 0::2] = np.sin(position * div_term)
    pe[:, 1::2] = np.cos(position * div_term)
    return pe


_PE = jnp.asarray(_positional_encoding(MAX_LEN, EMBED)[:SEQLEN])

_mesh = plsc.VectorSubcoreMesh(
    core_axis_name="c", subcore_axis_name="s", num_cores=NC, num_subcores=NS
)


@functools.partial(
    pl.kernel,
    out_type=jax.ShapeDtypeStruct((TOT, EMBED), jnp.float32),
    mesh=_mesh,
    scratch_types=[
        [pltpu.VMEM((NSUB, 128), jnp.int32)] * 2,
        [pltpu.VMEM((NSUB, 128), jnp.int32)] * 2,
        [pltpu.VMEM((NSUB, 128), jnp.int32)] * 2,
        [pltpu.VMEM((CHUNK, EMBED), jnp.float32)] * 2,
        [pltpu.SemaphoreType.DMA] * 2,
    ],
    compiler_params=pltpu.CompilerParams(use_tc_tiling_on_sc=False),
)
def _embed_kernel(seq_hbm, lbl_hbm, tok_hbm, c_hbm, out_hbm,
                  sidx, lidx, cidx, dbuf, sem):
    wid = lax.axis_index("s") * NC + lax.axis_index("c")
    wbase = wid * PER_W
    wrow = wid * (PER_W // 128)

    def drain_out(k):
        pltpu.make_async_copy(
            out_hbm.at[pl.ds(0, CHUNK)], dbuf[k], sem[k]
        ).wait()

    def start_chunk(ci, k):
        base = wbase + ci * CHUNK
        brow = wrow + ci * NSUB
        a = pltpu.async_copy(seq_hbm.at[pl.ds(brow, NSUB)], sidx[k], sem[k])
        b = pltpu.async_copy(lbl_hbm.at[pl.ds(brow, NSUB)], lidx[k], sem[k])
        a.wait()
        b.wait()
        toks = [
            pltpu.async_copy(
                tok_hbm.at[sidx[k].at[s]], dbuf[k].at[pl.ds(s * 128, 128)], sem[k]
            )
            for s in range(NSUB)
        ]
        for j in range(CHUNK // 16):
            s, c = divmod(j, 8)
            lane = lax.iota(jnp.int32, 16) + (base + j * 16)
            pos = lax.rem(lane, SEQLEN)
            cidx[k][s, pl.ds(c * 16, 16)] = pos * 3 + lidx[k][s, pl.ds(c * 16, 16)]
        return toks

    def start_add(toks, k):
        for cp in toks:
            cp.wait()
        return [
            pltpu.async_copy(
                c_hbm.at[cidx[k].at[s]], dbuf[k].at[pl.ds(s * 128, 128)],
                sem[k], add=True,
            )
            for s in range(NSUB)
        ]

    def start_out(ci, adds, k):
        for cp in adds:
            cp.wait()
        pltpu.async_copy(dbuf[k], out_hbm.at[pl.ds(wbase + ci * CHUNK, CHUNK)], sem[k])

    def pair_body(p, carry):
        ci0 = p * 2
        ci1 = ci0 + 1

        @pl.when(p != 0)
        def _():
            drain_out(0)

        toks0 = start_chunk(ci0, 0)

        @pl.when(p != 0)
        def _():
            drain_out(1)

        toks1 = start_chunk(ci1, 1)
        adds0 = start_add(toks0, 0)
        adds1 = start_add(toks1, 1)
        start_out(ci0, adds0, 0)
        start_out(ci1, adds1, 1)
        return carry

    lax.fori_loop(0, NCH // 2, pair_body, 0)
    drain_out(0)
    drain_out(1)


def kernel(sequence, segment_label, token_table, segment_table):
    seq2d = sequence.reshape(TOT // 128, 128).astype(jnp.int32)
    lbl2d = segment_label.reshape(TOT // 128, 128).astype(jnp.int32)
    c_tab = (_PE[:, None, :] + segment_table[None, :, :]).reshape(3 * SEQLEN, EMBED)
    out = _embed_kernel(seq2d, lbl2d, token_table, c_tab)
    return out.reshape(BATCH, SEQLEN, EMBED)
